# baseline (speedup 1.0000x reference)
import functools

import jax
import jax.numpy as jnp
from jax import lax
from jax.experimental import pallas as pl
from jax.experimental.pallas import tpu as pltpu
from jax.experimental.pallas import tpu_sc as plsc


def _make_sc_gather(num_rows, num_views, n_row_blocks, row_block):
    info = plsc.get_sparse_core_info()
    nc, ns = info.num_cores, info.num_subcores
    nw = nc * ns
    assert n_row_blocks % nw == 0
    blocks_per_w = n_row_blocks // nw

    mesh = plsc.VectorSubcoreMesh(core_axis_name="c", subcore_axis_name="s")

    @functools.partial(
        pl.kernel,
        out_type=jax.ShapeDtypeStruct(
            (n_row_blocks, row_block, num_views), jnp.float32),
        mesh=mesh,
        scratch_types=[
            pltpu.VMEM((blocks_per_w, row_block), jnp.int32),
            pltpu.VMEM((blocks_per_w, row_block, num_views), jnp.float32),
            pltpu.SemaphoreType.DMA,
        ],
        compiler_params=pltpu.CompilerParams(use_tc_tiling_on_sc=False),
    )
    def sc_gather(table_hbm, ids_hbm, out_hbm, idx_v, rows_v, sem):
        wid = lax.axis_index("s") * nc + lax.axis_index("c")
        base = wid * blocks_per_w
        pltpu.sync_copy(ids_hbm.at[pl.ds(base, blocks_per_w)], idx_v)
        copies = []
        for j in range(blocks_per_w):
            copies.append(
                pltpu.async_copy(table_hbm.at[idx_v.at[j]], rows_v.at[j], sem))
        for c in copies:
            c.wait()
        pltpu.sync_copy(rows_v, out_hbm.at[pl.ds(base, blocks_per_w)])

    return sc_gather


def _tc_fuse_body(logits_ref, img_ref, fused_ref, w_ref):
    lg = logits_ref[...]
    m = jnp.max(lg, axis=-1, keepdims=True)
    e = jnp.exp(lg - m)
    s = jnp.sum(e, axis=-1, keepdims=True)
    w = e / s
    w_ref[...] = w
    img = img_ref[...]
    fused_ref[...] = jnp.sum(w[:, :, None] * img, axis=1)


def kernel(img_views, subject_ids, view_logits_weight):
    b, k, d = img_views.shape
    num_rows = view_logits_weight.shape[0]

    row_block = 128
    n_row_blocks = b // row_block
    ids2 = subject_ids.astype(jnp.int32).reshape(n_row_blocks, row_block)

    gather = _make_sc_gather(num_rows, k, n_row_blocks, row_block)
    logits = gather(view_logits_weight, ids2).reshape(b, k)

    tb = 512
    grid = (b // tb,)
    fused, weights = pl.pallas_call(
        _tc_fuse_body,
        grid=grid,
        in_specs=[
            pl.BlockSpec((tb, k), lambda i: (i, 0)),
            pl.BlockSpec((tb, k, d), lambda i: (i, 0, 0)),
        ],
        out_specs=[
            pl.BlockSpec((tb, d), lambda i: (i, 0)),
            pl.BlockSpec((tb, k), lambda i: (i, 0)),
        ],
        out_shape=[
            jax.ShapeDtypeStruct((b, d), jnp.float32),
            jax.ShapeDtypeStruct((b, k), jnp.float32),
        ],
    )(logits, img_views)
    return (fused, weights)
